# Initial kernel scaffold; baseline (speedup 1.0000x reference)
import jax
import jax.numpy as jnp
from jax.experimental import pallas as pl


def kernel(h, u, U, V, bias):
    raise NotImplementedError("write your pallas kernel here")


# speedup vs baseline: 1.4508x; 1.4508x over previous
import jax
import jax.numpy as jnp
from jax.experimental import pallas as pl

B = 2048
D = 128
E = 64
R = 256
K = 8


def _gate_kernel(h_ref, u_ref, U_ref, V_ref, bias_ref, out_ref):
    e = pl.program_id(0)

    Ue = U_ref[0]
    Ve = V_ref[0]
    hU = jax.lax.dot_general(h_ref[...], Ue, (((1,), (1,)), ((), ())),
                             preferred_element_type=jnp.float32)
    uV = jax.lax.dot_general(u_ref[...], Ve, (((1,), (1,)), ((), ())),
                             preferred_element_type=jnp.float32)
    col = jnp.sum(hU * uV, axis=1, keepdims=True)

    @pl.when(e == 0)
    def _():
        out_ref[...] = jnp.zeros_like(out_ref)

    lane = jax.lax.broadcasted_iota(jnp.int32, (B, E), 1)
    g = jnp.where(lane == e, col, out_ref[...])

    @pl.when(e < E - 1)
    def _():
        out_ref[...] = g

    @pl.when(e == E - 1)
    def _():
        x = g + bias_ref[...]
        cur = x
        for _ in range(K - 1):
            m = jnp.max(cur, axis=1, keepdims=True)
            cur = jnp.where(cur >= m, -jnp.inf, cur)
        t8 = jnp.max(cur, axis=1, keepdims=True)
        sel = x >= t8
        xm = jnp.max(x, axis=1, keepdims=True)
        ex = jnp.where(sel, jnp.exp(x - xm), 0.0)
        out_ref[...] = ex / jnp.sum(ex, axis=1, keepdims=True)


def kernel(h, u, U, V, bias):
    bias2 = bias.reshape(1, E)
    return pl.pallas_call(
        _gate_kernel,
        grid=(E,),
        in_specs=[
            pl.BlockSpec((B, D), lambda e: (0, 0)),
            pl.BlockSpec((B, D), lambda e: (0, 0)),
            pl.BlockSpec((1, R, D), lambda e: (e, 0, 0)),
            pl.BlockSpec((1, R, D), lambda e: (e, 0, 0)),
            pl.BlockSpec((1, E), lambda e: (0, 0)),
        ],
        out_specs=pl.BlockSpec((B, E), lambda e: (0, 0)),
        out_shape=jax.ShapeDtypeStruct((B, E), jnp.float32),
    )(h, u, U, V, bias2)


# speedup vs baseline: 2.1873x; 1.5076x over previous
import jax
import jax.numpy as jnp
from jax.experimental import pallas as pl
from jax.experimental.pallas import tpu as pltpu

B = 2048
D = 128
E = 64
R = 256
K = 8
G = 4


def _gate_kernel(h_ref, u_ref, U_ref, V_ref, bias_ref, out_ref, g_ref):
    i = pl.program_id(0)
    h = h_ref[...]
    u = u_ref[...]

    for j in range(G):
        Ue = U_ref[j * R:(j + 1) * R, :]
        Ve = V_ref[j * R:(j + 1) * R, :]
        hU = jax.lax.dot_general(h, Ue, (((1,), (1,)), ((), ())),
                                 preferred_element_type=jnp.float32)
        uV = jax.lax.dot_general(u, Ve, (((1,), (1,)), ((), ())),
                                 preferred_element_type=jnp.float32)
        p = hU[:, :128] * uV[:, :128] + hU[:, 128:] * uV[:, 128:]
        pt = jax.lax.transpose(p, (1, 0))
        colT = jnp.sum(pt, axis=0, keepdims=True)
        g_ref[pl.ds(i * G + j, 1), :] = colT

    @pl.when(i == (E // G) - 1)
    def _():
        x = g_ref[...] + bias_ref[...]
        cur = x
        for _ in range(K - 1):
            m = jnp.max(cur, axis=0, keepdims=True)
            cur = jnp.where(cur >= m, -jnp.inf, cur)
        t8 = jnp.max(cur, axis=0, keepdims=True)
        sel = x >= t8
        xm = jnp.max(x, axis=0, keepdims=True)
        ex = jnp.where(sel, jnp.exp(x - xm), 0.0)
        w = ex / jnp.sum(ex, axis=0, keepdims=True)
        out_ref[...] = jax.lax.transpose(w, (1, 0))


def kernel(h, u, U, V, bias):
    Ur = U.reshape(E * R, D)
    Vr = V.reshape(E * R, D)
    bias2 = bias.reshape(E, 1)
    return pl.pallas_call(
        _gate_kernel,
        grid=(E // G,),
        in_specs=[
            pl.BlockSpec((B, D), lambda i: (0, 0)),
            pl.BlockSpec((B, D), lambda i: (0, 0)),
            pl.BlockSpec((G * R, D), lambda i: (i, 0)),
            pl.BlockSpec((G * R, D), lambda i: (i, 0)),
            pl.BlockSpec((E, 1), lambda i: (0, 0)),
        ],
        out_specs=pl.BlockSpec((B, E), lambda i: (0, 0)),
        out_shape=jax.ShapeDtypeStruct((B, E), jnp.float32),
        scratch_shapes=[pltpu.VMEM((E, B), jnp.float32)],
    )(h, u, Ur, Vr, bias2)


# speedup vs baseline: 2.2483x; 1.0279x over previous
import jax
import jax.numpy as jnp
from jax.experimental import pallas as pl
from jax.experimental.pallas import tpu as pltpu

B = 2048
D = 128
E = 64
R = 256
K = 8
G = 4


def _gate_kernel(h_ref, u_ref, U_ref, V_ref, bias_ref, out_ref, g_ref):
    i = pl.program_id(0)
    h = h_ref[...]
    u = u_ref[...]

    hUT = jax.lax.dot_general(U_ref[...], h, (((1,), (1,)), ((), ())),
                              preferred_element_type=jnp.float32)
    uVT = jax.lax.dot_general(V_ref[...], u, (((1,), (1,)), ((), ())),
                              preferred_element_type=jnp.float32)
    p = hUT * uVT
    for j in range(G):
        pj = p[j * R:(j + 1) * R, :]
        g_ref[pl.ds(i * G + j, 1), :] = jnp.sum(pj, axis=0, keepdims=True)

    @pl.when(i == (E // G) - 1)
    def _():
        x = g_ref[...] + bias_ref[...]
        cur = x
        for _ in range(K - 1):
            m = jnp.max(cur, axis=0, keepdims=True)
            cur = jnp.where(cur >= m, -jnp.inf, cur)
        t8 = jnp.max(cur, axis=0, keepdims=True)
        sel = x >= t8
        xm = jnp.max(x, axis=0, keepdims=True)
        ex = jnp.where(sel, jnp.exp(x - xm), 0.0)
        w = ex / jnp.sum(ex, axis=0, keepdims=True)
        out_ref[...] = jax.lax.transpose(w, (1, 0))


def kernel(h, u, U, V, bias):
    Ur = U.reshape(E * R, D)
    Vr = V.reshape(E * R, D)
    bias2 = bias.reshape(E, 1)
    return pl.pallas_call(
        _gate_kernel,
        grid=(E // G,),
        in_specs=[
            pl.BlockSpec((B, D), lambda i: (0, 0)),
            pl.BlockSpec((B, D), lambda i: (0, 0)),
            pl.BlockSpec((G * R, D), lambda i: (i, 0)),
            pl.BlockSpec((G * R, D), lambda i: (i, 0)),
            pl.BlockSpec((E, 1), lambda i: (0, 0)),
        ],
        out_specs=pl.BlockSpec((B, E), lambda i: (0, 0)),
        out_shape=jax.ShapeDtypeStruct((B, E), jnp.float32),
        scratch_shapes=[pltpu.VMEM((E, B), jnp.float32)],
    )(h, u, Ur, Vr, bias2)
